# baseline (speedup 1.0000x reference)
import functools

import jax
import jax.numpy as jnp
from jax import lax
from jax.experimental import pallas as pl
from jax.experimental.pallas import tpu as pltpu
from jax.experimental.pallas import tpu_sc as plsc

N = 10000
E = 160000
D = 256
S = 64

BLK = 1000
NBLK = N // BLK

NTILE = 16
EPT = E // NTILE
CH = 176
NCHUNK_T = EPT // CH
TAIL_E = EPT - NCHUNK_T * CH
ROWS_A = 624
ROWS_LAST = N - 15 * ROWS_A


def _embed_body(x_ref, w_ref, g_ref, b_ref, wx_ref, bg_ref,
                lo_ref, hi_ref, gx_ref):
    nr = jnp.tanh(jnp.dot(x_ref[...].astype(jnp.bfloat16),
                          w_ref[...].astype(jnp.bfloat16),
                          preferred_element_type=jnp.float32))
    m = jnp.mean(nr, axis=-1, keepdims=True)
    v = jnp.mean((nr - m) ** 2, axis=-1, keepdims=True)
    y = (nr - m) / jnp.sqrt(v + 1e-5) * g_ref[...] + b_ref[...]
    lo_ref[...] = y[:, :128]
    hi_ref[...] = y[:, 128:]
    gx_ref[...] = (jnp.dot(y.astype(jnp.bfloat16),
                           wx_ref[...].astype(jnp.bfloat16),
                           preferred_element_type=jnp.float32)
                   + bg_ref[...])


def _embed(node_feats, W_emb, ln_g, ln_b, W_x, b_g):
    return pl.pallas_call(
        _embed_body,
        grid=(NBLK,),
        in_specs=[
            pl.BlockSpec((BLK, D), lambda i: (i, 0)),
            pl.BlockSpec((D, D), lambda i: (0, 0)),
            pl.BlockSpec((1, D), lambda i: (0, 0)),
            pl.BlockSpec((1, D), lambda i: (0, 0)),
            pl.BlockSpec((D, 4 * D), lambda i: (0, 0)),
            pl.BlockSpec((1, 4 * D), lambda i: (0, 0)),
        ],
        out_specs=[
            pl.BlockSpec((BLK, 128), lambda i: (i, 0)),
            pl.BlockSpec((BLK, 128), lambda i: (i, 0)),
            pl.BlockSpec((BLK, 4 * D), lambda i: (i, 0)),
        ],
        out_shape=[
            jax.ShapeDtypeStruct((N, 128), jnp.float32),
            jax.ShapeDtypeStruct((N, 128), jnp.float32),
            jax.ShapeDtypeStruct((N, 4 * D), jnp.float32),
        ],
    )(node_feats, W_emb, ln_g.reshape(1, D), ln_b.reshape(1, D),
      W_x, b_g.reshape(1, 4 * D))


def _seg_sum(src, dst, nr_lo, nr_hi, zeros):
    mesh = plsc.VectorSubcoreMesh(core_axis_name="c", subcore_axis_name="s")

    @functools.partial(
        pl.kernel,
        mesh=mesh,
        out_type=[
            jax.ShapeDtypeStruct((N, 128), jnp.float32),
            jax.ShapeDtypeStruct((N, 128), jnp.float32),
        ],
        scratch_types=[
            pltpu.VMEM((CH,), jnp.int32),
            pltpu.VMEM((CH,), jnp.int32),
            pltpu.VMEM((CH,), jnp.int32),
            pltpu.VMEM((CH,), jnp.int32),
            pltpu.VMEM((CH,), jnp.int32),
            pltpu.VMEM((CH,), jnp.int32),
            pltpu.VMEM((CH,), jnp.int32),
            pltpu.VMEM((CH,), jnp.int32),
            pltpu.VMEM((CH, 128), jnp.float32),
            pltpu.VMEM((CH, 128), jnp.float32),
            pltpu.VMEM((TAIL_E,), jnp.int32),
            pltpu.VMEM((TAIL_E,), jnp.int32),
            pltpu.VMEM_SHARED((N, 128), jnp.float32),
            pltpu.SemaphoreType.DMA,
            pltpu.SemaphoreType.DMA,
            pltpu.SemaphoreType.DMA,
            pltpu.SemaphoreType.DMA,
            pltpu.SemaphoreType.DMA,
            pltpu.SemaphoreType.DMA,
        ],
    )
    def sc_kernel(src_hbm, dst_hbm, nrlo_hbm, nrhi_hbm, zero_hbm,
                  outlo_hbm, outhi_hbm, isrc0, isrc1, isrc2, isrc3,
                  idst0, idst1, idst2, idst3, rows0, rows1,
                  src_t, dst_t, acc, gsem0, gsem1, is0, is1, is2, is3):
        c = lax.axis_index("c")
        s = lax.axis_index("s")
        isrc = (isrc0, isrc1, isrc2, isrc3)
        idst = (idst0, idst1, idst2, idst3)
        rows = (rows0, rows1)
        gsems = (gsem0, gsem1)
        isems = (is0, is1, is2, is3)

        def half(nr_hbm, out_hbm):
            @pl.when(s < 15)
            def _():
                pltpu.sync_copy(zero_hbm.at[pl.ds(s * ROWS_A, ROWS_A)],
                                acc.at[pl.ds(s * ROWS_A, ROWS_A)])

            @pl.when(s == 15)
            def _():
                pltpu.sync_copy(zero_hbm.at[pl.ds(15 * ROWS_A, ROWS_LAST)],
                                acc.at[pl.ds(15 * ROWS_A, ROWS_LAST)])

            plsc.subcore_barrier()

            def i_copies(k, slot):
                base = s * EPT + k * CH
                return (
                    pltpu.make_async_copy(src_hbm.at[pl.ds(base, CH)],
                                          isrc[slot], isems[slot]),
                    pltpu.make_async_copy(dst_hbm.at[pl.ds(base, CH)],
                                          idst[slot], isems[slot]),
                )

            def i_start(k, slot):
                for cp in i_copies(k, slot):
                    cp.start()

            def i_wait(k, slot):
                for cp in i_copies(k, slot):
                    cp.wait()

            def g_start(slot, rslot):
                pltpu.make_async_copy(nr_hbm.at[isrc[slot]],
                                      rows[rslot], gsems[rslot]).start()

            def g_wait(slot, rslot):
                pltpu.make_async_copy(nr_hbm.at[isrc[slot]],
                                      rows[rslot], gsems[rslot]).wait()

            def s_add(slot, rslot):
                pltpu.sync_copy(rows[rslot], acc.at[idst[slot]], add=True)

            i_start(0, 0)
            i_start(1, 1)
            i_wait(0, 0)
            g_start(0, 0)

            @pl.loop(0, NCHUNK_T // 4)
            def _(t):
                for j in range(4):
                    k = 4 * t + j

                    @pl.when(k + 2 < NCHUNK_T)
                    def _():
                        i_start(k + 2, (j + 2) % 4)

                    @pl.when(k + 1 < NCHUNK_T)
                    def _():
                        i_wait(k + 1, (j + 1) % 4)
                        g_start((j + 1) % 4, (j + 1) % 2)

                    g_wait(j, j % 2)
                    s_add(j, j % 2)

            tbase = s * EPT + NCHUNK_T * CH
            pltpu.sync_copy(src_hbm.at[pl.ds(tbase, TAIL_E)], src_t)
            pltpu.sync_copy(dst_hbm.at[pl.ds(tbase, TAIL_E)], dst_t)
            pltpu.async_copy(nr_hbm.at[src_t],
                             rows0.at[pl.ds(0, TAIL_E)], gsem0).wait()
            pltpu.sync_copy(rows0.at[pl.ds(0, TAIL_E)],
                            acc.at[dst_t], add=True)

            plsc.subcore_barrier()

            @pl.when(s < 15)
            def _():
                pltpu.sync_copy(acc.at[pl.ds(s * ROWS_A, ROWS_A)],
                                out_hbm.at[pl.ds(s * ROWS_A, ROWS_A)])

            @pl.when(s == 15)
            def _():
                pltpu.sync_copy(acc.at[pl.ds(15 * ROWS_A, ROWS_LAST)],
                                out_hbm.at[pl.ds(15 * ROWS_A, ROWS_LAST)])

        @pl.when(c == 0)
        def _():
            half(nrlo_hbm, outlo_hbm)

        @pl.when(c == 1)
        def _():
            half(nrhi_hbm, outhi_hbm)

    return sc_kernel(src, dst, nr_lo, nr_hi, zeros)


def _cell_body(lo_b, hi_b, gx_ref, hlo_ref, hhi_ref,
               wh_ref, seg_ref, out_ref, acc_ref):
    i = pl.program_id(0)

    @pl.when(i == 0)
    def _():
        acc_ref[...] = jnp.full((S, D), -jnp.inf, jnp.float32)

    h = jnp.concatenate([hlo_ref[...], hhi_ref[...]], axis=1)
    gates = (gx_ref[...]
             + jnp.dot(h.astype(jnp.bfloat16),
                       wh_ref[...].astype(jnp.bfloat16),
                       preferred_element_type=jnp.float32))
    i_g = jax.nn.sigmoid(gates[:, :D])
    f_g = jax.nn.sigmoid(gates[:, D:2 * D])
    g_g = jnp.tanh(gates[:, 2 * D:3 * D])
    o_g = jax.nn.sigmoid(gates[:, 3 * D:])
    cell = i_g * g_g + f_g * h
    pooled = o_g * jnp.tanh(cell)

    seg = seg_ref[...]
    s_lo = lo_b[i]
    s_hi = hi_b[i]

    def body(sid, _):
        contrib = jnp.where(seg == sid, pooled, -jnp.inf)
        mx = jnp.max(contrib, axis=0)[None, :]
        acc_ref[pl.ds(sid, 1), :] = jnp.maximum(acc_ref[pl.ds(sid, 1), :], mx)
        return 0

    lax.fori_loop(s_lo, s_hi + 1, body, 0)

    @pl.when(i == NBLK - 1)
    def _():
        out_ref[...] = acc_ref[...]


def _cell_and_pool(gx, h_lo, h_hi, W_h, seg_col, blk_lo, blk_hi):
    return pl.pallas_call(
        _cell_body,
        grid=(NBLK,),
        in_specs=[
            pl.BlockSpec(memory_space=pltpu.SMEM),
            pl.BlockSpec(memory_space=pltpu.SMEM),
            pl.BlockSpec((BLK, 4 * D), lambda i: (i, 0)),
            pl.BlockSpec((BLK, 128), lambda i: (i, 0)),
            pl.BlockSpec((BLK, 128), lambda i: (i, 0)),
            pl.BlockSpec((D, 4 * D), lambda i: (0, 0)),
            pl.BlockSpec((BLK, 1), lambda i: (i, 0)),
        ],
        out_specs=pl.BlockSpec((S, D), lambda i: (0, 0)),
        out_shape=jax.ShapeDtypeStruct((S, D), jnp.float32),
        scratch_shapes=[pltpu.VMEM((S, D), jnp.float32)],
    )(blk_lo, blk_hi, gx, h_lo, h_hi, W_h, seg_col.reshape(N, 1))


def kernel(node_feats, edge_index, segment_ids, W_emb, ln_g, ln_b, W_x, W_h, b_g):
    src = edge_index[0].astype(jnp.int32)
    dst = edge_index[1].astype(jnp.int32)
    seg = segment_ids.astype(jnp.int32)

    nr_lo, nr_hi, gx = _embed(node_feats, W_emb, ln_g, ln_b, W_x, b_g)

    zeros = jnp.zeros((N, 128), jnp.float32)
    h_lo, h_hi = _seg_sum(src, dst, nr_lo, nr_hi, zeros)

    starts = jnp.arange(NBLK, dtype=jnp.int32) * BLK
    blk_lo = seg[starts]
    blk_hi = seg[starts + (BLK - 1)]

    return _cell_and_pool(gx, h_lo, h_hi, W_h, seg, blk_lo, blk_hi)


# speedup vs baseline: 1.0387x; 1.0387x over previous
import functools

import jax
import jax.numpy as jnp
from jax import lax
from jax.experimental import pallas as pl
from jax.experimental.pallas import tpu as pltpu
from jax.experimental.pallas import tpu_sc as plsc

N = 10000
E = 160000
D = 256
S = 64

BLK = 1000
NBLK = N // BLK

NTILE = 16
EPT = E // NTILE
CH = 176
NCHUNK_T = EPT // CH
TAIL_E = EPT - NCHUNK_T * CH
ROWS_A = 624
ROWS_LAST = N - 15 * ROWS_A


def _embed_body(x_ref, w_ref, g_ref, b_ref, lo_ref, hi_ref):
    nr = jnp.tanh(jnp.dot(x_ref[...].astype(jnp.bfloat16),
                          w_ref[...].astype(jnp.bfloat16),
                          preferred_element_type=jnp.float32))
    m = jnp.mean(nr, axis=-1, keepdims=True)
    v = jnp.mean((nr - m) ** 2, axis=-1, keepdims=True)
    y = (nr - m) / jnp.sqrt(v + 1e-5) * g_ref[...] + b_ref[...]
    lo_ref[...] = y[:, :128]
    hi_ref[...] = y[:, 128:]


def _embed(node_feats, W_emb, ln_g, ln_b):
    return pl.pallas_call(
        _embed_body,
        grid=(NBLK,),
        in_specs=[
            pl.BlockSpec((BLK, D), lambda i: (i, 0)),
            pl.BlockSpec((D, D), lambda i: (0, 0)),
            pl.BlockSpec((1, D), lambda i: (0, 0)),
            pl.BlockSpec((1, D), lambda i: (0, 0)),
        ],
        out_specs=[
            pl.BlockSpec((BLK, 128), lambda i: (i, 0)),
            pl.BlockSpec((BLK, 128), lambda i: (i, 0)),
        ],
        out_shape=[
            jax.ShapeDtypeStruct((N, 128), jnp.float32),
            jax.ShapeDtypeStruct((N, 128), jnp.float32),
        ],
    )(node_feats, W_emb, ln_g.reshape(1, D), ln_b.reshape(1, D))


def _seg_sum(src, dst, nr_lo, nr_hi, zeros):
    mesh = plsc.VectorSubcoreMesh(core_axis_name="c", subcore_axis_name="s")

    @functools.partial(
        pl.kernel,
        mesh=mesh,
        out_type=[
            jax.ShapeDtypeStruct((N, 128), jnp.float32),
            jax.ShapeDtypeStruct((N, 128), jnp.float32),
        ],
        scratch_types=[
            pltpu.VMEM((CH,), jnp.int32),
            pltpu.VMEM((CH,), jnp.int32),
            pltpu.VMEM((CH,), jnp.int32),
            pltpu.VMEM((CH,), jnp.int32),
            pltpu.VMEM((CH,), jnp.int32),
            pltpu.VMEM((CH,), jnp.int32),
            pltpu.VMEM((CH,), jnp.int32),
            pltpu.VMEM((CH,), jnp.int32),
            pltpu.VMEM((CH, 128), jnp.float32),
            pltpu.VMEM((CH, 128), jnp.float32),
            pltpu.VMEM((TAIL_E,), jnp.int32),
            pltpu.VMEM((TAIL_E,), jnp.int32),
            pltpu.VMEM_SHARED((N, 128), jnp.float32),
            pltpu.SemaphoreType.DMA,
            pltpu.SemaphoreType.DMA,
            pltpu.SemaphoreType.DMA,
            pltpu.SemaphoreType.DMA,
            pltpu.SemaphoreType.DMA,
            pltpu.SemaphoreType.DMA,
        ],
    )
    def sc_kernel(src_hbm, dst_hbm, nrlo_hbm, nrhi_hbm, zero_hbm,
                  outlo_hbm, outhi_hbm, isrc0, isrc1, isrc2, isrc3,
                  idst0, idst1, idst2, idst3, rows0, rows1,
                  src_t, dst_t, acc, gsem0, gsem1, is0, is1, is2, is3):
        c = lax.axis_index("c")
        s = lax.axis_index("s")
        isrc = (isrc0, isrc1, isrc2, isrc3)
        idst = (idst0, idst1, idst2, idst3)
        rows = (rows0, rows1)
        gsems = (gsem0, gsem1)
        isems = (is0, is1, is2, is3)

        def half(nr_hbm, out_hbm):
            @pl.when(s < 15)
            def _():
                pltpu.sync_copy(zero_hbm.at[pl.ds(s * ROWS_A, ROWS_A)],
                                acc.at[pl.ds(s * ROWS_A, ROWS_A)])

            @pl.when(s == 15)
            def _():
                pltpu.sync_copy(zero_hbm.at[pl.ds(15 * ROWS_A, ROWS_LAST)],
                                acc.at[pl.ds(15 * ROWS_A, ROWS_LAST)])

            plsc.subcore_barrier()

            def i_copies(k, slot):
                base = s * EPT + k * CH
                return (
                    pltpu.make_async_copy(src_hbm.at[pl.ds(base, CH)],
                                          isrc[slot], isems[slot]),
                    pltpu.make_async_copy(dst_hbm.at[pl.ds(base, CH)],
                                          idst[slot], isems[slot]),
                )

            def i_start(k, slot):
                for cp in i_copies(k, slot):
                    cp.start()

            def i_wait(k, slot):
                for cp in i_copies(k, slot):
                    cp.wait()

            def g_start(slot, rslot):
                pltpu.make_async_copy(nr_hbm.at[isrc[slot]],
                                      rows[rslot], gsems[rslot]).start()

            def g_wait(slot, rslot):
                pltpu.make_async_copy(nr_hbm.at[isrc[slot]],
                                      rows[rslot], gsems[rslot]).wait()

            def s_add(slot, rslot):
                pltpu.sync_copy(rows[rslot], acc.at[idst[slot]], add=True)

            i_start(0, 0)
            i_start(1, 1)
            i_wait(0, 0)
            g_start(0, 0)

            @pl.loop(0, NCHUNK_T // 4)
            def _(t):
                for j in range(4):
                    k = 4 * t + j

                    @pl.when(k + 2 < NCHUNK_T)
                    def _():
                        i_start(k + 2, (j + 2) % 4)

                    @pl.when(k + 1 < NCHUNK_T)
                    def _():
                        i_wait(k + 1, (j + 1) % 4)
                        g_start((j + 1) % 4, (j + 1) % 2)

                    g_wait(j, j % 2)
                    s_add(j, j % 2)

            tbase = s * EPT + NCHUNK_T * CH
            pltpu.sync_copy(src_hbm.at[pl.ds(tbase, TAIL_E)], src_t)
            pltpu.sync_copy(dst_hbm.at[pl.ds(tbase, TAIL_E)], dst_t)
            pltpu.async_copy(nr_hbm.at[src_t],
                             rows0.at[pl.ds(0, TAIL_E)], gsem0).wait()
            pltpu.sync_copy(rows0.at[pl.ds(0, TAIL_E)],
                            acc.at[dst_t], add=True)

            plsc.subcore_barrier()

            @pl.when(s < 15)
            def _():
                pltpu.sync_copy(acc.at[pl.ds(s * ROWS_A, ROWS_A)],
                                out_hbm.at[pl.ds(s * ROWS_A, ROWS_A)])

            @pl.when(s == 15)
            def _():
                pltpu.sync_copy(acc.at[pl.ds(15 * ROWS_A, ROWS_LAST)],
                                out_hbm.at[pl.ds(15 * ROWS_A, ROWS_LAST)])

        @pl.when(c == 0)
        def _():
            half(nrlo_hbm, outlo_hbm)

        @pl.when(c == 1)
        def _():
            half(nrhi_hbm, outhi_hbm)

    return sc_kernel(src, dst, nr_lo, nr_hi, zeros)


def _cell_body(lo_b, hi_b, nrlo_ref, nrhi_ref, hlo_ref, hhi_ref,
               wx_ref, wh_ref, bg_ref, seg_ref, out_ref, acc_ref):
    i = pl.program_id(0)

    @pl.when(i == 0)
    def _():
        acc_ref[...] = jnp.full((S, D), -jnp.inf, jnp.float32)

    nr = jnp.concatenate([nrlo_ref[...], nrhi_ref[...]], axis=1)
    h = jnp.concatenate([hlo_ref[...], hhi_ref[...]], axis=1)
    gates = (jnp.dot(nr.astype(jnp.bfloat16),
                     wx_ref[...].astype(jnp.bfloat16),
                     preferred_element_type=jnp.float32)
             + jnp.dot(h.astype(jnp.bfloat16),
                       wh_ref[...].astype(jnp.bfloat16),
                       preferred_element_type=jnp.float32)
             + bg_ref[...])
    i_g = jax.nn.sigmoid(gates[:, :D])
    f_g = jax.nn.sigmoid(gates[:, D:2 * D])
    g_g = jnp.tanh(gates[:, 2 * D:3 * D])
    o_g = jax.nn.sigmoid(gates[:, 3 * D:])
    cell = i_g * g_g + f_g * h
    pooled = o_g * jnp.tanh(cell)

    seg = seg_ref[...]
    s_lo = lo_b[i]
    s_hi = hi_b[i]

    def body(sid, _):
        contrib = jnp.where(seg == sid, pooled, -jnp.inf)
        mx = jnp.max(contrib, axis=0)[None, :]
        acc_ref[pl.ds(sid, 1), :] = jnp.maximum(acc_ref[pl.ds(sid, 1), :], mx)
        return 0

    lax.fori_loop(s_lo, s_hi + 1, body, 0)

    @pl.when(i == NBLK - 1)
    def _():
        out_ref[...] = acc_ref[...]


def _cell_and_pool(nr_lo, nr_hi, h_lo, h_hi, W_x, W_h, b_g,
                   seg_col, blk_lo, blk_hi):
    return pl.pallas_call(
        _cell_body,
        grid=(NBLK,),
        in_specs=[
            pl.BlockSpec(memory_space=pltpu.SMEM),
            pl.BlockSpec(memory_space=pltpu.SMEM),
            pl.BlockSpec((BLK, 128), lambda i: (i, 0)),
            pl.BlockSpec((BLK, 128), lambda i: (i, 0)),
            pl.BlockSpec((BLK, 128), lambda i: (i, 0)),
            pl.BlockSpec((BLK, 128), lambda i: (i, 0)),
            pl.BlockSpec((D, 4 * D), lambda i: (0, 0)),
            pl.BlockSpec((D, 4 * D), lambda i: (0, 0)),
            pl.BlockSpec((1, 4 * D), lambda i: (0, 0)),
            pl.BlockSpec((BLK, 1), lambda i: (i, 0)),
        ],
        out_specs=pl.BlockSpec((S, D), lambda i: (0, 0)),
        out_shape=jax.ShapeDtypeStruct((S, D), jnp.float32),
        scratch_shapes=[pltpu.VMEM((S, D), jnp.float32)],
    )(blk_lo, blk_hi, nr_lo, nr_hi, h_lo, h_hi, W_x, W_h,
      b_g.reshape(1, 4 * D), seg_col.reshape(N, 1))


def kernel(node_feats, edge_index, segment_ids, W_emb, ln_g, ln_b, W_x, W_h, b_g):
    src = edge_index[0].astype(jnp.int32)
    dst = edge_index[1].astype(jnp.int32)
    seg = segment_ids.astype(jnp.int32)

    nr_lo, nr_hi = _embed(node_feats, W_emb, ln_g, ln_b)

    zeros = jnp.zeros((N, 128), jnp.float32)
    h_lo, h_hi = _seg_sum(src, dst, nr_lo, nr_hi, zeros)

    starts = jnp.arange(NBLK, dtype=jnp.int32) * BLK
    blk_lo = seg[starts]
    blk_hi = seg[starts + (BLK - 1)]

    return _cell_and_pool(nr_lo, nr_hi, h_lo, h_hi, W_x, W_h, b_g,
                          seg, blk_lo, blk_hi)
